# baseline (speedup 1.0000x reference)
import functools

import jax
import jax.numpy as jnp
from jax import lax
from jax.experimental import pallas as pl
from jax.experimental.pallas import tpu as pltpu
from jax.experimental.pallas import tpu_sc as plsc

_N = 10000
_NP = 10240
_F = 128
_NC = 2
_NS = 16
_NW = _NC * _NS
_K = 128
_C = 80
_EP = _NW * _C * _K
_RPT = _NP // _NS


def _make_sc_pass(width):
  mesh = plsc.VectorSubcoreMesh(
      core_axis_name="c", subcore_axis_name="s",
      num_cores=_NC, num_subcores=_NS)
  if width == 1:
    out_shape = (_NC, _NP)
    rows_shape = (_K,)
    acc_shape = (_NP,)
  else:
    out_shape = (_NC, _NP, width)
    rows_shape = (_K, width)
    acc_shape = (_NP, width)

  def body(srcb, dstb, zeros, table, out, srcv, dstv0, dstv1, rows0, rows1,
           acc, sem0, sem1):
    ci = lax.axis_index("c")
    si = lax.axis_index("s")
    wid = ci * _NS + si
    lo = si * _RPT
    pltpu.sync_copy(zeros.at[pl.ds(lo, _RPT)], acc.at[pl.ds(lo, _RPT)])
    pltpu.sync_copy(srcb.at[wid], srcv)
    plsc.subcore_barrier()

    pltpu.async_copy(table.at[srcv.at[0]], rows0, sem0)
    pltpu.async_copy(dstb.at[wid, 0], dstv0, sem0)
    pltpu.async_copy(table.at[srcv.at[1]], rows1, sem1)
    pltpu.async_copy(dstb.at[wid, 1], dstv1, sem1)

    def pair(p, carry):
      c0 = 2 * p
      c1 = c0 + 1
      pltpu.make_async_copy(table.at[srcv.at[c0]], rows0, sem0).wait()
      pltpu.make_async_copy(dstb.at[wid, c0], dstv0, sem0).wait()
      pltpu.sync_copy(rows0, acc.at[dstv0], add=True)
      pltpu.async_copy(table.at[srcv.at[(c0 + 2) % _C]], rows0, sem0)
      pltpu.async_copy(dstb.at[wid, (c0 + 2) % _C], dstv0, sem0)
      pltpu.make_async_copy(table.at[srcv.at[c1]], rows1, sem1).wait()
      pltpu.make_async_copy(dstb.at[wid, c1], dstv1, sem1).wait()
      pltpu.sync_copy(rows1, acc.at[dstv1], add=True)
      pltpu.async_copy(table.at[srcv.at[(c1 + 2) % _C]], rows1, sem1)
      pltpu.async_copy(dstb.at[wid, (c1 + 2) % _C], dstv1, sem1)
      return carry

    lax.fori_loop(0, _C // 2, pair, 0)
    pltpu.make_async_copy(table.at[srcv.at[0]], rows0, sem0).wait()
    pltpu.make_async_copy(dstb.at[wid, 0], dstv0, sem0).wait()
    pltpu.make_async_copy(table.at[srcv.at[1]], rows1, sem1).wait()
    pltpu.make_async_copy(dstb.at[wid, 1], dstv1, sem1).wait()
    plsc.subcore_barrier()
    pltpu.sync_copy(acc.at[pl.ds(lo, _RPT)], out.at[ci, pl.ds(lo, _RPT)])

  return pl.kernel(
      body,
      out_type=jax.ShapeDtypeStruct(out_shape, jnp.float32),
      mesh=mesh,
      scratch_types=[
          pltpu.VMEM((_C, _K), jnp.int32),
          pltpu.VMEM((_K,), jnp.int32),
          pltpu.VMEM((_K,), jnp.int32),
          pltpu.VMEM(rows_shape, jnp.float32),
          pltpu.VMEM(rows_shape, jnp.float32),
          pltpu.VMEM_SHARED(acc_shape, jnp.float32),
          pltpu.SemaphoreType.DMA,
          pltpu.SemaphoreType.DMA,
      ])


_sc_pass_wF = _make_sc_pass(_F)

_C2 = 2 * _C
_SL = _NP // _NS
_NV = _SL // 16


def _rsqrt_nr(d):
  y = 1.0 / d
  for _ in range(20):
    y = y * (1.5 - 0.5 * d * y * y)
  return y


def _fused_scalar_body(srcb, dstb, zeros, xin, w1in, w2in, b2in,
                       dinv_out, h2s_out,
                       srcv, dstv, ra0, ra1, rb0, rb1, ones_v,
                       xbuf, dgbuf, dibuf, tbuf, tabuf, tbbuf,
                       w2buf, abuf, bbuf, hbuf0, hbuf1,
                       deg_sh, xs_sh, t_sh, ta_sh, tb_sh, uacc_sh, vacc_sh,
                       sa0, sa1, sb0, sb1):
  ci = lax.axis_index("c")
  si = lax.axis_index("s")

  @pl.when(ci == 0)
  def _():
    lo = si * _SL

    pltpu.sync_copy(srcb.at[si], srcv)
    pltpu.sync_copy(dstb.at[si], dstv)
    pltpu.sync_copy(zeros.at[pl.ds(lo, _SL)], deg_sh.at[pl.ds(lo, _SL)])
    pltpu.sync_copy(zeros.at[pl.ds(lo, _SL)], t_sh.at[pl.ds(lo, _SL)])
    pltpu.sync_copy(zeros.at[pl.ds(lo, _SL)], uacc_sh.at[pl.ds(lo, _SL)])
    pltpu.sync_copy(zeros.at[pl.ds(lo, _SL)], vacc_sh.at[pl.ds(lo, _SL)])
    pltpu.sync_copy(xin.at[pl.ds(lo, _SL)], xbuf)

    def fill_ones(i, c):
      ones_v[pl.ds(i * 16, 16)] = jnp.full((16,), 1.0, jnp.float32)
      return c

    lax.fori_loop(0, _K // 16, fill_ones, 0)
    plsc.subcore_barrier()

    pltpu.async_copy(ones_v, deg_sh.at[dstv.at[0]], sa0, add=True)
    pltpu.async_copy(ones_v, deg_sh.at[dstv.at[1]], sa1, add=True)

    def deg_pair(p, c):
      c0 = 2 * p
      pltpu.make_async_copy(ones_v, deg_sh.at[dstv.at[c0]], sa0).wait()
      pltpu.async_copy(ones_v, deg_sh.at[dstv.at[(c0 + 2) % _C2]], sa0,
                       add=True)
      pltpu.make_async_copy(ones_v, deg_sh.at[dstv.at[c0 + 1]], sa1).wait()
      pltpu.async_copy(ones_v, deg_sh.at[dstv.at[(c0 + 3) % _C2]], sa1,
                       add=True)
      return c

    lax.fori_loop(0, _C2 // 2 - 1, deg_pair, 0)
    c0 = _C2 - 2
    pltpu.make_async_copy(ones_v, deg_sh.at[dstv.at[c0]], sa0).wait()
    pltpu.make_async_copy(ones_v, deg_sh.at[dstv.at[c0 + 1]], sa1).wait()
    plsc.subcore_barrier()

    pltpu.sync_copy(deg_sh.at[pl.ds(lo, _SL)], dgbuf)

    def ew1(i, c):
      sl = pl.ds(i * 16, 16)
      y = _rsqrt_nr(dgbuf[sl] + 1.0)
      dibuf[sl] = y
      dgbuf[sl] = y * xbuf[sl]
      return c

    lax.fori_loop(0, _NV, ew1, 0)
    pltpu.sync_copy(dibuf.at[pl.ds(0, _SL)], dinv_out.at[pl.ds(lo, _SL)])
    pltpu.sync_copy(dgbuf, xs_sh.at[pl.ds(lo, _SL)])
    plsc.subcore_barrier()

    pltpu.async_copy(xs_sh.at[srcv.at[0]], ra0, sa0)
    pltpu.async_copy(xs_sh.at[srcv.at[1]], ra1, sa1)

    def s_pair(p, c):
      c0 = 2 * p
      pltpu.make_async_copy(xs_sh.at[srcv.at[c0]], ra0, sa0).wait()
      pltpu.sync_copy(ra0, t_sh.at[dstv.at[c0]], add=True)
      pltpu.async_copy(xs_sh.at[srcv.at[(c0 + 2) % _C2]], ra0, sa0)
      pltpu.make_async_copy(xs_sh.at[srcv.at[c0 + 1]], ra1, sa1).wait()
      pltpu.sync_copy(ra1, t_sh.at[dstv.at[c0 + 1]], add=True)
      pltpu.async_copy(xs_sh.at[srcv.at[(c0 + 3) % _C2]], ra1, sa1)
      return c

    lax.fori_loop(0, _C2 // 2, s_pair, 0)
    pltpu.make_async_copy(xs_sh.at[srcv.at[0]], ra0, sa0).wait()
    pltpu.make_async_copy(xs_sh.at[srcv.at[1]], ra1, sa1).wait()
    plsc.subcore_barrier()

    pltpu.sync_copy(t_sh.at[pl.ds(lo, _SL)], tbuf)

    def ew2(i, c):
      sl = pl.ds(i * 16, 16)
      y = dibuf[sl]
      s = y * tbuf[sl] + y * y * xbuf[sl]
      tabuf[sl] = y * jnp.maximum(s, 0.0)
      tbbuf[sl] = y * jnp.maximum(-s, 0.0)
      return c

    lax.fori_loop(0, _NV, ew2, 0)
    pltpu.sync_copy(tabuf.at[pl.ds(0, _SL)], ta_sh.at[pl.ds(lo, _SL)])
    pltpu.sync_copy(tbbuf.at[pl.ds(0, _SL)], tb_sh.at[pl.ds(lo, _SL)])
    plsc.subcore_barrier()

    pltpu.async_copy(ta_sh.at[srcv.at[0]], ra0, sa0)
    pltpu.async_copy(tb_sh.at[srcv.at[0]], rb0, sb0)
    pltpu.async_copy(ta_sh.at[srcv.at[1]], ra1, sa1)
    pltpu.async_copy(tb_sh.at[srcv.at[1]], rb1, sb1)

    def uv_pair(p, c):
      c0 = 2 * p
      pltpu.make_async_copy(ta_sh.at[srcv.at[c0]], ra0, sa0).wait()
      pltpu.sync_copy(ra0, uacc_sh.at[dstv.at[c0]], add=True)
      pltpu.async_copy(ta_sh.at[srcv.at[(c0 + 2) % _C2]], ra0, sa0)
      pltpu.make_async_copy(tb_sh.at[srcv.at[c0]], rb0, sb0).wait()
      pltpu.sync_copy(rb0, vacc_sh.at[dstv.at[c0]], add=True)
      pltpu.async_copy(tb_sh.at[srcv.at[(c0 + 2) % _C2]], rb0, sb0)
      pltpu.make_async_copy(ta_sh.at[srcv.at[c0 + 1]], ra1, sa1).wait()
      pltpu.sync_copy(ra1, uacc_sh.at[dstv.at[c0 + 1]], add=True)
      pltpu.async_copy(ta_sh.at[srcv.at[(c0 + 3) % _C2]], ra1, sa1)
      pltpu.make_async_copy(tb_sh.at[srcv.at[c0 + 1]], rb1, sb1).wait()
      pltpu.sync_copy(rb1, vacc_sh.at[dstv.at[c0 + 1]], add=True)
      pltpu.async_copy(tb_sh.at[srcv.at[(c0 + 3) % _C2]], rb1, sb1)
      return c

    lax.fori_loop(0, _C2 // 2, uv_pair, 0)
    pltpu.make_async_copy(ta_sh.at[srcv.at[0]], ra0, sa0).wait()
    pltpu.make_async_copy(tb_sh.at[srcv.at[0]], rb0, sb0).wait()
    pltpu.make_async_copy(ta_sh.at[srcv.at[1]], ra1, sa1).wait()
    pltpu.make_async_copy(tb_sh.at[srcv.at[1]], rb1, sb1).wait()
    plsc.subcore_barrier()

    pltpu.sync_copy(uacc_sh.at[pl.ds(lo, _SL)], tbuf)
    pltpu.sync_copy(vacc_sh.at[pl.ds(lo, _SL)], xbuf)

    def ew3(i, c):
      sl = pl.ds(i * 16, 16)
      y = dibuf[sl]
      tabuf[sl] = y * tbuf[sl] + y * tabuf[sl]
      tbbuf[sl] = y * xbuf[sl] + y * tbbuf[sl]
      return c

    lax.fori_loop(0, _NV, ew3, 0)

    pltpu.sync_copy(w2in, w2buf)
    pltpu.sync_copy(w1in, dgbuf.at[pl.ds(0, 64)])
    pltpu.sync_copy(b2in, xbuf.at[pl.ds(0, _F)])
    for k in range(_F // 16):
      abuf[pl.ds(k * 16, 16)] = jnp.full((16,), 0.0, jnp.float32)
      bbuf[pl.ds(k * 16, 16)] = jnp.full((16,), 0.0, jnp.float32)

    def acc_ab(j, c):
      w = dgbuf[pl.ds(j, 16)][0]
      wp = jnp.maximum(w, 0.0)
      wm = jnp.maximum(-w, 0.0)
      for k in range(_F // 16):
        sl = pl.ds(k * 16, 16)
        wsl = pl.ds(j * _F + k * 16, 16)
        abuf[sl] = abuf[sl] + wp * w2buf[wsl]
        bbuf[sl] = bbuf[sl] + wm * w2buf[wsl]
      return c

    lax.fori_loop(0, 64, acc_ab, 0)

    def fill(buf, b):
      def node(i, c):
        g = 64 * b + i
        uu = tabuf[pl.ds(g, 16)][0]
        vv = tbbuf[pl.ds(g, 16)][0]
        dd = dibuf[pl.ds(g, 16)][0]
        for k in range(_F // 16):
          sl = pl.ds(k * 16, 16)
          buf[pl.ds(i * _F + k * 16, 16)] = dd * jnp.maximum(
              uu * abuf[sl] + vv * bbuf[sl] + xbuf[sl], 0.0)
        return c

      lax.fori_loop(0, 64, node, 0)

    blk = 64 * _F

    def hslice(b):
      return h2s_out.at[pl.ds((lo + b * 64) * _F, blk)]

    fill(hbuf0, 0)
    pltpu.async_copy(hbuf0, hslice(0), sa0)

    def pairblk(p, c):
      b0 = 2 * p
      fill(hbuf1, b0 + 1)
      pltpu.async_copy(hbuf1, hslice(b0 + 1), sa1)
      pltpu.make_async_copy(hbuf0, hslice(0), sa0).wait()
      fill(hbuf0, b0 + 2)
      pltpu.async_copy(hbuf0, hslice(b0 + 2), sa0)
      pltpu.make_async_copy(hbuf1, hslice(1), sa1).wait()
      return c

    lax.fori_loop(0, _SL // 128 - 1, pairblk, 0)
    fill(hbuf1, _SL // 64 - 1)
    pltpu.async_copy(hbuf1, hslice(_SL // 64 - 1), sa1)
    pltpu.make_async_copy(hbuf0, hslice(0), sa0).wait()
    pltpu.make_async_copy(hbuf1, hslice(1), sa1).wait()


_fused_scalar = pl.kernel(
    _fused_scalar_body,
    out_type=(jax.ShapeDtypeStruct((_NP,), jnp.float32),
              jax.ShapeDtypeStruct((_NP * _F,), jnp.float32)),
    mesh=plsc.VectorSubcoreMesh(core_axis_name="c", subcore_axis_name="s",
                                num_cores=_NC, num_subcores=_NS),
    scratch_types=[
        pltpu.VMEM((_C2, _K), jnp.int32),
        pltpu.VMEM((_C2, _K), jnp.int32),
        pltpu.VMEM((_K,), jnp.float32),
        pltpu.VMEM((_K,), jnp.float32),
        pltpu.VMEM((_K,), jnp.float32),
        pltpu.VMEM((_K,), jnp.float32),
        pltpu.VMEM((_K,), jnp.float32),
        pltpu.VMEM((_SL,), jnp.float32),
        pltpu.VMEM((_SL,), jnp.float32),
        pltpu.VMEM((_SL + 16,), jnp.float32),
        pltpu.VMEM((_SL,), jnp.float32),
        pltpu.VMEM((_SL + 16,), jnp.float32),
        pltpu.VMEM((_SL + 16,), jnp.float32),
        pltpu.VMEM((64 * _F,), jnp.float32),
        pltpu.VMEM((_F,), jnp.float32),
        pltpu.VMEM((_F,), jnp.float32),
        pltpu.VMEM((64 * _F,), jnp.float32),
        pltpu.VMEM((64 * _F,), jnp.float32),
        pltpu.VMEM_SHARED((_NP,), jnp.float32),
        pltpu.VMEM_SHARED((_NP,), jnp.float32),
        pltpu.VMEM_SHARED((_NP,), jnp.float32),
        pltpu.VMEM_SHARED((_NP,), jnp.float32),
        pltpu.VMEM_SHARED((_NP,), jnp.float32),
        pltpu.VMEM_SHARED((_NP,), jnp.float32),
        pltpu.VMEM_SHARED((_NP,), jnp.float32),
        pltpu.SemaphoreType.DMA,
        pltpu.SemaphoreType.DMA,
        pltpu.SemaphoreType.DMA,
        pltpu.SemaphoreType.DMA,
    ])


_OUTER = (((0,), (0,)), ((), ()))


def _final_body(a0, a1, h2s, dinv, w3, b3, f1w, f1b, f2w, f2b, out_ref, gacc):
  k = pl.program_id(0)

  @pl.when(k == 0)
  def _():
    gacc[...] = jnp.zeros_like(gacc)

  hi = lax.Precision.HIGHEST
  dv = dinv[0]
  dd = lax.dot_general(dv, jnp.ones((1, _F), jnp.float32), _OUTER,
                       precision=hi, preferred_element_type=jnp.float32)
  p2 = dd * (a0[...] + a1[...] + h2s[...])
  h3 = jnp.maximum(
      lax.dot_general(p2, w3[...], (((1,), (0,)), ((), ())),
                      precision=hi, preferred_element_type=jnp.float32)
      + b3[...], 0.0)
  gi = 128 * k + lax.broadcasted_iota(jnp.int32, (128, _F), 0)
  h3 = jnp.where(gi < _N, h3, 0.0)
  gacc[...] += jnp.sum(h3, axis=0, keepdims=True)

  @pl.when(k == (_NP // 128) - 1)
  def _():
    g = gacc[...] * (1.0 / _N)
    z = jnp.maximum(
        lax.dot_general(g, f1w[...], (((1,), (0,)), ((), ())),
                        precision=hi, preferred_element_type=jnp.float32)
        + f1b[...], 0.0)
    y = lax.dot_general(z, f2w[...], (((1,), (0,)), ((), ())),
                        precision=hi, preferred_element_type=jnp.float32) \
        + f2b[...]
    out_ref[...] = jax.nn.sigmoid(y)


def kernel(x, edge_index, W1, b1, W2, b2, W3, b3, fc1_W, fc1_b, fc2_W, fc2_b):
  f32 = jnp.float32
  src = edge_index[0]
  dst = edge_index[1]
  e = src.shape[0]
  npad = _EP - e
  padidx = (_N + (jnp.arange(npad, dtype=jnp.int32) % (_NP - _N))).astype(jnp.int32)
  srcp = jnp.concatenate([src, padidx]).reshape(_NW, _C, _K)
  dstp = jnp.concatenate([dst, padidx]).reshape(_NW, _C, _K)

  xp = jnp.pad(x[:, 0], (0, _NP - _N))
  z1 = jnp.zeros((_NP,), f32)

  dinv1, h2sf = _fused_scalar(
      srcp.reshape(_NS, _C2, _K), dstp.reshape(_NS, _C2, _K), z1, xp,
      W1.reshape(64), W2.reshape(64 * _F), b2)
  h2s = h2sf.reshape(_NP, _F)
  row3 = lambda a: a.reshape(_NP // 128, 1, 128)
  row_spec = pl.BlockSpec((1, 1, 128), lambda k: (k, 0, 0))

  zF = jnp.zeros((_NP, _F), f32)
  accpart = _sc_pass_wF(srcp, dstp, zF, h2s)

  blk_spec = pl.BlockSpec((128, _F), lambda k: (k, 0))
  out2 = pl.pallas_call(
      _final_body,
      grid=(_NP // 128,),
      out_shape=jax.ShapeDtypeStruct((1, 1), f32),
      in_specs=[blk_spec, blk_spec, blk_spec, row_spec,
                pl.BlockSpec((_F, _F), lambda k: (0, 0)),
                pl.BlockSpec((1, _F), lambda k: (0, 0)),
                pl.BlockSpec((_F, 64), lambda k: (0, 0)),
                pl.BlockSpec((1, 64), lambda k: (0, 0)),
                pl.BlockSpec((64, 1), lambda k: (0, 0)),
                pl.BlockSpec((1, 1), lambda k: (0, 0))],
      out_specs=pl.BlockSpec((1, 1), lambda k: (0, 0)),
      scratch_shapes=[pltpu.VMEM((1, _F), f32)],
  )(accpart[0], accpart[1], h2s, row3(dinv1), W3, b3.reshape(1, _F),
    fc1_W, fc1_b.reshape(1, 64), fc2_W, fc2_b.reshape(1, 1))

  return out2.reshape((1,))


# speedup vs baseline: 1.1142x; 1.1142x over previous
import functools

import jax
import jax.numpy as jnp
from jax import lax
from jax.experimental import pallas as pl
from jax.experimental.pallas import tpu as pltpu
from jax.experimental.pallas import tpu_sc as plsc

_N = 10000
_NP = 10240
_F = 128
_NC = 2
_NS = 16
_NW = _NC * _NS
_K = 128
_C = 80
_EP = _NW * _C * _K
_RPT = _NP // _NS


def _make_sc_pass(width):
  mesh = plsc.VectorSubcoreMesh(
      core_axis_name="c", subcore_axis_name="s",
      num_cores=_NC, num_subcores=_NS)
  if width == 1:
    out_shape = (_NC, _NP)
    rows_shape = (_K,)
    acc_shape = (_NP,)
  else:
    out_shape = (_NC, _NP, width)
    rows_shape = (_K, width)
    acc_shape = (_NP, width)

  def body(srcb, dstb, zeros, table, out, srcv, dstv0, dstv1, rows0, rows1,
           acc, sem0, sem1):
    ci = lax.axis_index("c")
    si = lax.axis_index("s")
    wid = ci * _NS + si
    lo = si * _RPT
    pltpu.sync_copy(zeros.at[pl.ds(lo, _RPT)], acc.at[pl.ds(lo, _RPT)])
    pltpu.sync_copy(srcb.at[wid], srcv)
    plsc.subcore_barrier()

    pltpu.async_copy(table.at[srcv.at[0]], rows0, sem0)
    pltpu.async_copy(dstb.at[wid, 0], dstv0, sem0)
    pltpu.async_copy(table.at[srcv.at[1]], rows1, sem1)
    pltpu.async_copy(dstb.at[wid, 1], dstv1, sem1)

    def pair(p, carry):
      c0 = 2 * p
      c1 = c0 + 1
      pltpu.make_async_copy(table.at[srcv.at[c0]], rows0, sem0).wait()
      pltpu.make_async_copy(dstb.at[wid, c0], dstv0, sem0).wait()
      pltpu.sync_copy(rows0, acc.at[dstv0], add=True)
      pltpu.async_copy(table.at[srcv.at[(c0 + 2) % _C]], rows0, sem0)
      pltpu.async_copy(dstb.at[wid, (c0 + 2) % _C], dstv0, sem0)
      pltpu.make_async_copy(table.at[srcv.at[c1]], rows1, sem1).wait()
      pltpu.make_async_copy(dstb.at[wid, c1], dstv1, sem1).wait()
      pltpu.sync_copy(rows1, acc.at[dstv1], add=True)
      pltpu.async_copy(table.at[srcv.at[(c1 + 2) % _C]], rows1, sem1)
      pltpu.async_copy(dstb.at[wid, (c1 + 2) % _C], dstv1, sem1)
      return carry

    lax.fori_loop(0, _C // 2, pair, 0)
    pltpu.make_async_copy(table.at[srcv.at[0]], rows0, sem0).wait()
    pltpu.make_async_copy(dstb.at[wid, 0], dstv0, sem0).wait()
    pltpu.make_async_copy(table.at[srcv.at[1]], rows1, sem1).wait()
    pltpu.make_async_copy(dstb.at[wid, 1], dstv1, sem1).wait()
    plsc.subcore_barrier()
    pltpu.sync_copy(acc.at[pl.ds(lo, _RPT)], out.at[ci, pl.ds(lo, _RPT)])

  return pl.kernel(
      body,
      out_type=jax.ShapeDtypeStruct(out_shape, jnp.float32),
      mesh=mesh,
      scratch_types=[
          pltpu.VMEM((_C, _K), jnp.int32),
          pltpu.VMEM((_K,), jnp.int32),
          pltpu.VMEM((_K,), jnp.int32),
          pltpu.VMEM(rows_shape, jnp.float32),
          pltpu.VMEM(rows_shape, jnp.float32),
          pltpu.VMEM_SHARED(acc_shape, jnp.float32),
          pltpu.SemaphoreType.DMA,
          pltpu.SemaphoreType.DMA,
      ])


_sc_pass_wF = _make_sc_pass(_F)

_C2 = 2 * _C
_SL = _NP // _NS
_NV = _SL // 16


def _rsqrt_nr(d):
  y = 1.0 / d
  for _ in range(20):
    y = y * (1.5 - 0.5 * d * y * y)
  return y


def _fused_scalar_body(srcb, dstb, zeros, xin, w1in, w2in, b2in,
                       dinv_out, h2s_out,
                       srcv, dstv, ra0, ra1, rb0, rb1, ones_v,
                       xbuf, dgbuf, dibuf, tbuf, tabuf, tbbuf,
                       w2buf, abuf, bbuf, hbuf0, hbuf1,
                       deg_sh, xs_sh, t_sh, ta_sh, tb_sh, uacc_sh, vacc_sh,
                       sa0, sa1, sb0, sb1):
  ci = lax.axis_index("c")
  si = lax.axis_index("s")

  @pl.when(ci == 0)
  def _():
    lo = si * _SL

    pltpu.sync_copy(srcb.at[si], srcv)
    pltpu.sync_copy(dstb.at[si], dstv)
    pltpu.sync_copy(zeros.at[pl.ds(lo, _SL)], deg_sh.at[pl.ds(lo, _SL)])
    pltpu.sync_copy(zeros.at[pl.ds(lo, _SL)], t_sh.at[pl.ds(lo, _SL)])
    pltpu.sync_copy(zeros.at[pl.ds(lo, _SL)], uacc_sh.at[pl.ds(lo, _SL)])
    pltpu.sync_copy(zeros.at[pl.ds(lo, _SL)], vacc_sh.at[pl.ds(lo, _SL)])
    pltpu.sync_copy(xin.at[pl.ds(lo, _SL)], xbuf)

    def fill_ones(i, c):
      ones_v[pl.ds(i * 16, 16)] = jnp.full((16,), 1.0, jnp.float32)
      return c

    lax.fori_loop(0, _K // 16, fill_ones, 0)
    plsc.subcore_barrier()

    pltpu.async_copy(ones_v, deg_sh.at[dstv.at[0]], sa0, add=True)
    pltpu.async_copy(ones_v, deg_sh.at[dstv.at[1]], sa1, add=True)

    def deg_pair(p, c):
      c0 = 2 * p
      pltpu.make_async_copy(ones_v, deg_sh.at[dstv.at[c0]], sa0).wait()
      pltpu.async_copy(ones_v, deg_sh.at[dstv.at[(c0 + 2) % _C2]], sa0,
                       add=True)
      pltpu.make_async_copy(ones_v, deg_sh.at[dstv.at[c0 + 1]], sa1).wait()
      pltpu.async_copy(ones_v, deg_sh.at[dstv.at[(c0 + 3) % _C2]], sa1,
                       add=True)
      return c

    lax.fori_loop(0, _C2 // 2 - 1, deg_pair, 0)
    c0 = _C2 - 2
    pltpu.make_async_copy(ones_v, deg_sh.at[dstv.at[c0]], sa0).wait()
    pltpu.make_async_copy(ones_v, deg_sh.at[dstv.at[c0 + 1]], sa1).wait()
    plsc.subcore_barrier()

    pltpu.sync_copy(deg_sh.at[pl.ds(lo, _SL)], dgbuf)

    def ew1(i, c):
      sl = pl.ds(i * 16, 16)
      y = _rsqrt_nr(dgbuf[sl] + 1.0)
      dibuf[sl] = y
      dgbuf[sl] = y * xbuf[sl]
      return c

    lax.fori_loop(0, _NV, ew1, 0)
    pltpu.sync_copy(dibuf.at[pl.ds(0, _SL)], dinv_out.at[pl.ds(lo, _SL)])
    pltpu.sync_copy(dgbuf, xs_sh.at[pl.ds(lo, _SL)])
    plsc.subcore_barrier()

    pltpu.async_copy(xs_sh.at[srcv.at[0]], ra0, sa0)
    pltpu.async_copy(xs_sh.at[srcv.at[1]], ra1, sa1)

    def s_pair(p, c):
      c0 = 2 * p
      pltpu.make_async_copy(xs_sh.at[srcv.at[c0]], ra0, sa0).wait()
      pltpu.sync_copy(ra0, t_sh.at[dstv.at[c0]], add=True)
      pltpu.async_copy(xs_sh.at[srcv.at[(c0 + 2) % _C2]], ra0, sa0)
      pltpu.make_async_copy(xs_sh.at[srcv.at[c0 + 1]], ra1, sa1).wait()
      pltpu.sync_copy(ra1, t_sh.at[dstv.at[c0 + 1]], add=True)
      pltpu.async_copy(xs_sh.at[srcv.at[(c0 + 3) % _C2]], ra1, sa1)
      return c

    lax.fori_loop(0, _C2 // 2, s_pair, 0)
    pltpu.make_async_copy(xs_sh.at[srcv.at[0]], ra0, sa0).wait()
    pltpu.make_async_copy(xs_sh.at[srcv.at[1]], ra1, sa1).wait()
    plsc.subcore_barrier()

    pltpu.sync_copy(t_sh.at[pl.ds(lo, _SL)], tbuf)

    def ew2(i, c):
      sl = pl.ds(i * 16, 16)
      y = dibuf[sl]
      s = y * tbuf[sl] + y * y * xbuf[sl]
      tabuf[sl] = y * jnp.maximum(s, 0.0)
      tbbuf[sl] = y * jnp.maximum(-s, 0.0)
      return c

    lax.fori_loop(0, _NV, ew2, 0)
    pltpu.sync_copy(tabuf.at[pl.ds(0, _SL)], ta_sh.at[pl.ds(lo, _SL)])
    pltpu.sync_copy(tbbuf.at[pl.ds(0, _SL)], tb_sh.at[pl.ds(lo, _SL)])
    plsc.subcore_barrier()

    pltpu.async_copy(ta_sh.at[srcv.at[0]], ra0, sa0)
    pltpu.async_copy(tb_sh.at[srcv.at[0]], rb0, sb0)
    pltpu.async_copy(ta_sh.at[srcv.at[1]], ra1, sa1)
    pltpu.async_copy(tb_sh.at[srcv.at[1]], rb1, sb1)

    def uv_pair(p, c):
      c0 = 2 * p
      pltpu.make_async_copy(ta_sh.at[srcv.at[c0]], ra0, sa0).wait()
      pltpu.sync_copy(ra0, uacc_sh.at[dstv.at[c0]], add=True)
      pltpu.async_copy(ta_sh.at[srcv.at[(c0 + 2) % _C2]], ra0, sa0)
      pltpu.make_async_copy(tb_sh.at[srcv.at[c0]], rb0, sb0).wait()
      pltpu.sync_copy(rb0, vacc_sh.at[dstv.at[c0]], add=True)
      pltpu.async_copy(tb_sh.at[srcv.at[(c0 + 2) % _C2]], rb0, sb0)
      pltpu.make_async_copy(ta_sh.at[srcv.at[c0 + 1]], ra1, sa1).wait()
      pltpu.sync_copy(ra1, uacc_sh.at[dstv.at[c0 + 1]], add=True)
      pltpu.async_copy(ta_sh.at[srcv.at[(c0 + 3) % _C2]], ra1, sa1)
      pltpu.make_async_copy(tb_sh.at[srcv.at[c0 + 1]], rb1, sb1).wait()
      pltpu.sync_copy(rb1, vacc_sh.at[dstv.at[c0 + 1]], add=True)
      pltpu.async_copy(tb_sh.at[srcv.at[(c0 + 3) % _C2]], rb1, sb1)
      return c

    lax.fori_loop(0, _C2 // 2, uv_pair, 0)
    pltpu.make_async_copy(ta_sh.at[srcv.at[0]], ra0, sa0).wait()
    pltpu.make_async_copy(tb_sh.at[srcv.at[0]], rb0, sb0).wait()
    pltpu.make_async_copy(ta_sh.at[srcv.at[1]], ra1, sa1).wait()
    pltpu.make_async_copy(tb_sh.at[srcv.at[1]], rb1, sb1).wait()
    plsc.subcore_barrier()

    pltpu.sync_copy(uacc_sh.at[pl.ds(lo, _SL)], tbuf)
    pltpu.sync_copy(vacc_sh.at[pl.ds(lo, _SL)], xbuf)

    def ew3(i, c):
      sl = pl.ds(i * 16, 16)
      y = dibuf[sl]
      tabuf[sl] = y * tbuf[sl] + y * tabuf[sl]
      tbbuf[sl] = y * xbuf[sl] + y * tbbuf[sl]
      return c

    lax.fori_loop(0, _NV, ew3, 0)

    pltpu.sync_copy(w2in, w2buf)
    pltpu.sync_copy(w1in, dgbuf.at[pl.ds(0, 64)])
    pltpu.sync_copy(b2in, xbuf.at[pl.ds(0, _F)])
    for k in range(_F // 16):
      abuf[pl.ds(k * 16, 16)] = jnp.full((16,), 0.0, jnp.float32)
      bbuf[pl.ds(k * 16, 16)] = jnp.full((16,), 0.0, jnp.float32)

    def acc_ab(j, c):
      w = dgbuf[pl.ds(j, 16)][0]
      wp = jnp.maximum(w, 0.0)
      wm = jnp.maximum(-w, 0.0)
      for k in range(_F // 16):
        sl = pl.ds(k * 16, 16)
        wsl = pl.ds(j * _F + k * 16, 16)
        abuf[sl] = abuf[sl] + wp * w2buf[wsl]
        bbuf[sl] = bbuf[sl] + wm * w2buf[wsl]
      return c

    lax.fori_loop(0, 64, acc_ab, 0)

    alpha_v = [abuf[pl.ds(k * 16, 16)] for k in range(_F // 16)]
    beta_v = [bbuf[pl.ds(k * 16, 16)] for k in range(_F // 16)]
    b2_v = [xbuf[pl.ds(k * 16, 16)] for k in range(_F // 16)]

    def fill(buf, b):
      def node8(q, c):
        for dq in range(8):
          i = 8 * q + dq
          g = 64 * b + i
          uu = tabuf[pl.ds(g, 16)][0]
          vv = tbbuf[pl.ds(g, 16)][0]
          dd = dibuf[pl.ds(g, 16)][0]
          for k in range(_F // 16):
            buf[pl.ds(i * _F + k * 16, 16)] = dd * jnp.maximum(
                uu * alpha_v[k] + vv * beta_v[k] + b2_v[k], 0.0)
        return c

      lax.fori_loop(0, 8, node8, 0)

    blk = 64 * _F

    def hslice(b):
      return h2s_out.at[pl.ds((lo + b * 64) * _F, blk)]

    fill(hbuf0, 0)
    pltpu.async_copy(hbuf0, hslice(0), sa0)

    def pairblk(p, c):
      b0 = 2 * p
      fill(hbuf1, b0 + 1)
      pltpu.async_copy(hbuf1, hslice(b0 + 1), sa1)
      pltpu.make_async_copy(hbuf0, hslice(0), sa0).wait()
      fill(hbuf0, b0 + 2)
      pltpu.async_copy(hbuf0, hslice(b0 + 2), sa0)
      pltpu.make_async_copy(hbuf1, hslice(1), sa1).wait()
      return c

    lax.fori_loop(0, _SL // 128 - 1, pairblk, 0)
    fill(hbuf1, _SL // 64 - 1)
    pltpu.async_copy(hbuf1, hslice(_SL // 64 - 1), sa1)
    pltpu.make_async_copy(hbuf0, hslice(0), sa0).wait()
    pltpu.make_async_copy(hbuf1, hslice(1), sa1).wait()


_fused_scalar = pl.kernel(
    _fused_scalar_body,
    out_type=(jax.ShapeDtypeStruct((_NP,), jnp.float32),
              jax.ShapeDtypeStruct((_NP * _F,), jnp.float32)),
    mesh=plsc.VectorSubcoreMesh(core_axis_name="c", subcore_axis_name="s",
                                num_cores=_NC, num_subcores=_NS),
    scratch_types=[
        pltpu.VMEM((_C2, _K), jnp.int32),
        pltpu.VMEM((_C2, _K), jnp.int32),
        pltpu.VMEM((_K,), jnp.float32),
        pltpu.VMEM((_K,), jnp.float32),
        pltpu.VMEM((_K,), jnp.float32),
        pltpu.VMEM((_K,), jnp.float32),
        pltpu.VMEM((_K,), jnp.float32),
        pltpu.VMEM((_SL,), jnp.float32),
        pltpu.VMEM((_SL,), jnp.float32),
        pltpu.VMEM((_SL + 16,), jnp.float32),
        pltpu.VMEM((_SL,), jnp.float32),
        pltpu.VMEM((_SL + 16,), jnp.float32),
        pltpu.VMEM((_SL + 16,), jnp.float32),
        pltpu.VMEM((64 * _F,), jnp.float32),
        pltpu.VMEM((_F,), jnp.float32),
        pltpu.VMEM((_F,), jnp.float32),
        pltpu.VMEM((64 * _F,), jnp.float32),
        pltpu.VMEM((64 * _F,), jnp.float32),
        pltpu.VMEM_SHARED((_NP,), jnp.float32),
        pltpu.VMEM_SHARED((_NP,), jnp.float32),
        pltpu.VMEM_SHARED((_NP,), jnp.float32),
        pltpu.VMEM_SHARED((_NP,), jnp.float32),
        pltpu.VMEM_SHARED((_NP,), jnp.float32),
        pltpu.VMEM_SHARED((_NP,), jnp.float32),
        pltpu.VMEM_SHARED((_NP,), jnp.float32),
        pltpu.SemaphoreType.DMA,
        pltpu.SemaphoreType.DMA,
        pltpu.SemaphoreType.DMA,
        pltpu.SemaphoreType.DMA,
    ])


_OUTER = (((0,), (0,)), ((), ()))


def _final_body(a0, a1, h2s, dinv, w3, b3, f1w, f1b, f2w, f2b, out_ref, gacc):
  k = pl.program_id(0)

  @pl.when(k == 0)
  def _():
    gacc[...] = jnp.zeros_like(gacc)

  hi = lax.Precision.HIGHEST
  dv = dinv[0]
  dd = lax.dot_general(dv, jnp.ones((1, _F), jnp.float32), _OUTER,
                       precision=hi, preferred_element_type=jnp.float32)
  p2 = dd * (a0[...] + a1[...] + h2s[...])
  h3 = jnp.maximum(
      lax.dot_general(p2, w3[...], (((1,), (0,)), ((), ())),
                      precision=hi, preferred_element_type=jnp.float32)
      + b3[...], 0.0)
  gi = 128 * k + lax.broadcasted_iota(jnp.int32, (128, _F), 0)
  h3 = jnp.where(gi < _N, h3, 0.0)
  gacc[...] += jnp.sum(h3, axis=0, keepdims=True)

  @pl.when(k == (_NP // 128) - 1)
  def _():
    g = gacc[...] * (1.0 / _N)
    z = jnp.maximum(
        lax.dot_general(g, f1w[...], (((1,), (0,)), ((), ())),
                        precision=hi, preferred_element_type=jnp.float32)
        + f1b[...], 0.0)
    y = lax.dot_general(z, f2w[...], (((1,), (0,)), ((), ())),
                        precision=hi, preferred_element_type=jnp.float32) \
        + f2b[...]
    out_ref[...] = jax.nn.sigmoid(y)


def kernel(x, edge_index, W1, b1, W2, b2, W3, b3, fc1_W, fc1_b, fc2_W, fc2_b):
  f32 = jnp.float32
  src = edge_index[0]
  dst = edge_index[1]
  e = src.shape[0]
  npad = _EP - e
  padidx = (_N + (jnp.arange(npad, dtype=jnp.int32) % (_NP - _N))).astype(jnp.int32)
  srcp = jnp.concatenate([src, padidx]).reshape(_NW, _C, _K)
  dstp = jnp.concatenate([dst, padidx]).reshape(_NW, _C, _K)

  xp = jnp.pad(x[:, 0], (0, _NP - _N))
  z1 = jnp.zeros((_NP,), f32)

  dinv1, h2sf = _fused_scalar(
      srcp.reshape(_NS, _C2, _K), dstp.reshape(_NS, _C2, _K), z1, xp,
      W1.reshape(64), W2.reshape(64 * _F), b2)
  h2s = h2sf.reshape(_NP, _F)
  row3 = lambda a: a.reshape(_NP // 128, 1, 128)
  row_spec = pl.BlockSpec((1, 1, 128), lambda k: (k, 0, 0))

  zF = jnp.zeros((_NP, _F), f32)
  accpart = _sc_pass_wF(srcp, dstp, zF, h2s)

  blk_spec = pl.BlockSpec((128, _F), lambda k: (k, 0))
  out2 = pl.pallas_call(
      _final_body,
      grid=(_NP // 128,),
      out_shape=jax.ShapeDtypeStruct((1, 1), f32),
      in_specs=[blk_spec, blk_spec, blk_spec, row_spec,
                pl.BlockSpec((_F, _F), lambda k: (0, 0)),
                pl.BlockSpec((1, _F), lambda k: (0, 0)),
                pl.BlockSpec((_F, 64), lambda k: (0, 0)),
                pl.BlockSpec((1, 64), lambda k: (0, 0)),
                pl.BlockSpec((64, 1), lambda k: (0, 0)),
                pl.BlockSpec((1, 1), lambda k: (0, 0))],
      out_specs=pl.BlockSpec((1, 1), lambda k: (0, 0)),
      scratch_shapes=[pltpu.VMEM((1, _F), f32)],
  )(accpart[0], accpart[1], h2s, row3(dinv1), W3, b3.reshape(1, _F),
    fc1_W, fc1_b.reshape(1, 64), fc2_W, fc2_b.reshape(1, 1))

  return out2.reshape((1,))
